# Initial kernel scaffold; baseline (speedup 1.0000x reference)
import jax
import jax.numpy as jnp
from jax.experimental import pallas as pl


def kernel(x, edge_index, edge_attr, tabular_features, Wl1, Wr1, b1, Wl2, Wr2, b2, g1, be1, g2, be2, W1c, b1c, W2c, b2c, W3c, b3c):
    raise NotImplementedError("write your pallas kernel here")


# speedup vs baseline: 4.1680x; 4.1680x over previous
import functools

import jax
import jax.numpy as jnp
from jax import lax
from jax.experimental import pallas as pl
from jax.experimental.pallas import tpu as pltpu
from jax.experimental.pallas import tpu_sc as plsc

N = 10000
E = 320000
D = 128
H = 64
W = 128

NC = 2
NS = 16
NW = NC * NS
EPW = E // NW
CE = 200
NCHUNK = EPW // CE
CG = 400
NGCHUNK = EPW // CG
NP = 10240
RPT = NP // NS
ZR = 160

_f32 = jnp.float32


def _sc_mesh():
    return plsc.VectorSubcoreMesh(core_axis_name="c", subcore_axis_name="s")


@functools.partial(
    pl.kernel,
    out_type=[jax.ShapeDtypeStruct((NC, NP, W), _f32)],
    mesh=_sc_mesh(),
    scratch_types=[
        pltpu.VMEM((CE,), jnp.int32),
        pltpu.VMEM((CE,), jnp.int32),
        pltpu.VMEM((CE, W), _f32),
        pltpu.VMEM_SHARED((NP, W), _f32),
        pltpu.SemaphoreType.DMA,
    ],
)
def _segsum(t_hbm, src_hbm, dst_hbm, zeros_hbm, sum_out,
            src_v, dst_v, rows_v, acc, sem):
    c = lax.axis_index("c")
    s = lax.axis_index("s")
    wid = c * NS + s

    pltpu.sync_copy(zeros_hbm, rows_v.at[pl.ds(0, ZR)])
    for j in range(RPT // ZR):
        pltpu.sync_copy(rows_v.at[pl.ds(0, ZR)],
                        acc.at[pl.ds(s * RPT + j * ZR, ZR)])
    plsc.subcore_barrier()

    def chunk(i, carry):
        base = wid * EPW + i * CE
        pltpu.sync_copy(src_hbm.at[pl.ds(base, CE)], src_v)
        pltpu.sync_copy(dst_hbm.at[pl.ds(base, CE)], dst_v)
        pltpu.async_copy(t_hbm.at[src_v], rows_v, sem).wait()
        pltpu.sync_copy(rows_v, acc.at[dst_v], add=True)
        return carry

    lax.fori_loop(0, NCHUNK, chunk, 0)
    plsc.subcore_barrier()

    for j in range(RPT // ZR):
        pltpu.sync_copy(acc.at[pl.ds(s * RPT + j * ZR, ZR)],
                        rows_v.at[pl.ds(0, ZR)])
        pltpu.sync_copy(rows_v.at[pl.ds(0, ZR)],
                        sum_out.at[c, pl.ds(s * RPT + j * ZR, ZR)])


@functools.partial(
    pl.kernel,
    out_type=[jax.ShapeDtypeStruct((E, W), _f32),
              jax.ShapeDtypeStruct((E, W), _f32)],
    mesh=_sc_mesh(),
    scratch_types=[
        pltpu.VMEM((CG,), jnp.int32),
        pltpu.VMEM((CG,), jnp.int32),
        pltpu.VMEM((CG, W), _f32),
        pltpu.VMEM((CG, W), _f32),
        pltpu.SemaphoreType.DMA,
        pltpu.SemaphoreType.DMA,
    ],
)
def _edge_gather(a_hbm, b_hbm, src_hbm, dst_hbm, as_out, bd_out,
                 src_v, dst_v, rows_s, rows_d, sem_s, sem_d):
    c = lax.axis_index("c")
    s = lax.axis_index("s")
    wid = c * NS + s

    def chunk(i, carry):
        base = wid * EPW + i * CG
        pltpu.sync_copy(src_hbm.at[pl.ds(base, CG)], src_v)
        pltpu.sync_copy(dst_hbm.at[pl.ds(base, CG)], dst_v)
        cp_s = pltpu.async_copy(a_hbm.at[src_v], rows_s, sem_s)
        cp_d = pltpu.async_copy(b_hbm.at[dst_v], rows_d, sem_d)
        cp_s.wait()
        pltpu.sync_copy(rows_s, as_out.at[pl.ds(base, CG)])
        cp_d.wait()
        pltpu.sync_copy(rows_d, bd_out.at[pl.ds(base, CG)])
        return carry

    lax.fori_loop(0, NGCHUNK, chunk, 0)


def _proj_body(x_ref, wl_ref, wr_ref, b_ref, t_ref, r_ref):
    xx = x_ref[...]
    p = jnp.dot(xx, wl_ref[...], preferred_element_type=_f32)
    t_ref[...] = jnp.concatenate([p, jnp.ones((N, H), _f32)], axis=-1)
    r_ref[...] = (jnp.dot(xx, wr_ref[...], preferred_element_type=_f32)
                  + b_ref[...])


def _bn_relu(pre, g, be):
    mu = jnp.mean(pre, axis=0, keepdims=True)
    var = jnp.mean((pre - mu) ** 2, axis=0, keepdims=True)
    hh = g * (pre - mu) / jnp.sqrt(var + 1e-5) + be
    return jnp.maximum(hh, 0.0)


def _combine1_body(s_ref, r_ref, g_ref, be_ref, wl_ref, wr_ref, b2_ref,
                   t2_ref, r2_ref, dinv_ref):
    deg = s_ref[0, 0:N, H:H + 1] + s_ref[1, 0:N, H:H + 1]
    dinv = 1.0 / jnp.maximum(deg, 1.0)
    agg = (s_ref[0, 0:N, 0:H] + s_ref[1, 0:N, 0:H]) * dinv
    hh = _bn_relu(agg + r_ref[...], g_ref[...], be_ref[...])
    p2 = jnp.dot(hh, wl_ref[...], preferred_element_type=_f32)
    t2_ref[...] = jnp.concatenate([p2, jnp.ones((N, H), _f32)], axis=-1)
    r2_ref[...] = (jnp.dot(hh, wr_ref[...], preferred_element_type=_f32)
                   + b2_ref[...])
    dinv_ref[...] = dinv


def _combine2_body(s_ref, dinv_ref, r_ref, g_ref, be_ref, w1s_ref,
                   w1d_ref, a_ref, b_ref):
    agg = (s_ref[0, 0:N, 0:H] + s_ref[1, 0:N, 0:H]) * dinv_ref[...]
    hh = _bn_relu(agg + r_ref[...], g_ref[...], be_ref[...])
    a_ref[...] = jnp.dot(hh, w1s_ref[...], preferred_element_type=_f32)
    b_ref[...] = jnp.dot(hh, w1d_ref[...], preferred_element_type=_f32)


def _edge_mlp_body(as_ref, bd_ref, ea_ref, tb_ref, w1e_ref, w1t_ref,
                   b1_ref, w2_ref, b2_ref, w3_ref, b3_ref, out_ref):
    z = as_ref[...] + bd_ref[...]
    z = z + jnp.dot(ea_ref[...], w1e_ref[...], preferred_element_type=_f32)
    z = z + jnp.dot(tb_ref[...], w1t_ref[...], preferred_element_type=_f32)
    z = jnp.maximum(z + b1_ref[...], 0.0)
    z = jnp.maximum(
        jnp.dot(z, w2_ref[...], preferred_element_type=_f32) + b2_ref[...],
        0.0)
    out_ref[...] = (jnp.dot(z, w3_ref[...], preferred_element_type=_f32)
                    + b3_ref[...])


def kernel(x, edge_index, edge_attr, tabular_features, Wl1, Wr1, b1,
           Wl2, Wr2, b2, g1, be1, g2, be2, W1c, b1c, W2c, b2c, W3c, b3c):
    src = edge_index[0]
    dst = edge_index[1]
    zeros_zr = jnp.zeros((ZR, W), _f32)

    t1, r1 = pl.pallas_call(
        _proj_body,
        out_shape=[jax.ShapeDtypeStruct((N, W), _f32),
                   jax.ShapeDtypeStruct((N, H), _f32)],
    )(x, Wl1, Wr1, b1.reshape(1, H))
    (s1,) = _segsum(t1, src, dst, zeros_zr)
    t2, r2, dinv = pl.pallas_call(
        _combine1_body,
        out_shape=[jax.ShapeDtypeStruct((N, W), _f32),
                   jax.ShapeDtypeStruct((N, H), _f32),
                   jax.ShapeDtypeStruct((N, 1), _f32)],
    )(s1, r1, g1.reshape(1, H), be1.reshape(1, H), Wl2, Wr2,
      b2.reshape(1, H))

    (s2,) = _segsum(t2, src, dst, zeros_zr)
    a_tab, b_tab = pl.pallas_call(
        _combine2_body,
        out_shape=[jax.ShapeDtypeStruct((N, W), _f32),
                   jax.ShapeDtypeStruct((N, W), _f32)],
    )(s2, dinv, r2, g2.reshape(1, H), be2.reshape(1, H),
      W1c[0:H], W1c[H:2 * H])

    a_s, b_d = _edge_gather(a_tab, b_tab, src, dst)

    w1e = W1c[2 * H:2 * H + 16]
    w1t = W1c[2 * H + 16:]
    BE = 4000
    c_out = W3c.shape[1]
    full = lambda shp: pl.BlockSpec(shp, lambda i: (0, 0))
    out = pl.pallas_call(
        _edge_mlp_body,
        grid=(E // BE,),
        in_specs=[
            pl.BlockSpec((BE, W), lambda i: (i, 0)),
            pl.BlockSpec((BE, W), lambda i: (i, 0)),
            pl.BlockSpec((BE, 16), lambda i: (i, 0)),
            pl.BlockSpec((BE, 8), lambda i: (i, 0)),
            full((16, 128)), full((8, 128)),
            full((1, 128)), full((128, H)), full((1, H)),
            full((H, c_out)), full((1, c_out)),
        ],
        out_specs=pl.BlockSpec((BE, c_out), lambda i: (i, 0)),
        out_shape=jax.ShapeDtypeStruct((E, c_out), _f32),
    )(a_s, b_d, edge_attr, tabular_features, w1e, w1t,
      b1c.reshape(1, 128), W2c, b2c.reshape(1, H), W3c,
      b3c.reshape(1, c_out))
    return out


# speedup vs baseline: 4.6952x; 1.1265x over previous
import functools

import jax
import jax.numpy as jnp
from jax import lax
from jax.experimental import pallas as pl
from jax.experimental.pallas import tpu as pltpu
from jax.experimental.pallas import tpu_sc as plsc

N = 10000
E = 320000
D = 128
H = 64
W = 128

NC = 2
NS = 16
NW = NC * NS
EPW = E // NW
CE = 160
NPAIR = 31
CT = 80
CG = 200
NGCHUNK = EPW // CG
NP = 10240
RPT = NP // NS

_f32 = jnp.float32
_bf16 = jnp.bfloat16


def _sc_mesh():
    return plsc.VectorSubcoreMesh(core_axis_name="c", subcore_axis_name="s")


@functools.partial(
    pl.kernel,
    out_type=[jax.ShapeDtypeStruct((NC, NP, W), _f32)],
    mesh=_sc_mesh(),
    scratch_types=[
        pltpu.VMEM((CE,), jnp.int32), pltpu.VMEM((CE,), jnp.int32),
        pltpu.VMEM((CE,), jnp.int32), pltpu.VMEM((CE,), jnp.int32),
        pltpu.VMEM((CT,), jnp.int32), pltpu.VMEM((CT,), jnp.int32),
        pltpu.VMEM((CE, W), _f32), pltpu.VMEM((CE, W), _f32),
        pltpu.VMEM_SHARED((NP, W), _f32),
        pltpu.SemaphoreType.DMA, pltpu.SemaphoreType.DMA,
        pltpu.SemaphoreType.DMA, pltpu.SemaphoreType.DMA,
    ],
)
def _segsum(t_hbm, src_hbm, dst_hbm, zeros_hbm, sum_out,
            src0, dst0, src1, dst1, src_t, dst_t, rows0, rows1,
            acc, g0, g1, s0, s1):
    c = lax.axis_index("c")
    s = lax.axis_index("s")
    wid = c * NS + s
    ebase = wid * EPW

    pltpu.sync_copy(zeros_hbm, rows0.at[pl.ds(0, CE)])
    for j in range(RPT // CE):
        pltpu.sync_copy(rows0.at[pl.ds(0, CE)],
                        acc.at[pl.ds(s * RPT + j * CE, CE)])
    plsc.subcore_barrier()

    pltpu.sync_copy(src_hbm.at[pl.ds(ebase, CE)], src0)
    pltpu.sync_copy(dst_hbm.at[pl.ds(ebase, CE)], dst0)
    pltpu.async_copy(t_hbm.at[src0], rows0, g0)

    def pair(i2, carry):
        base_b = ebase + (2 * i2 + 1) * CE
        pltpu.sync_copy(src_hbm.at[pl.ds(base_b, CE)], src1)
        pltpu.sync_copy(dst_hbm.at[pl.ds(base_b, CE)], dst1)
        pltpu.make_async_copy(t_hbm.at[src0], rows0, g0).wait()
        pltpu.async_copy(rows0, acc.at[dst0], s0, add=True)
        pltpu.async_copy(t_hbm.at[src1], rows1, g1)
        pltpu.make_async_copy(rows0, acc.at[dst0], s0).wait()

        @pl.when(i2 < NPAIR - 1)
        def _():
            base_a = ebase + (2 * i2 + 2) * CE
            pltpu.sync_copy(src_hbm.at[pl.ds(base_a, CE)], src0)
            pltpu.sync_copy(dst_hbm.at[pl.ds(base_a, CE)], dst0)
            pltpu.async_copy(t_hbm.at[src0], rows0, g0)

        pltpu.make_async_copy(t_hbm.at[src1], rows1, g1).wait()
        pltpu.async_copy(rows1, acc.at[dst1], s1, add=True)
        pltpu.make_async_copy(rows1, acc.at[dst1], s1).wait()
        return carry

    lax.fori_loop(0, NPAIR, pair, 0)

    tbase = ebase + 2 * NPAIR * CE
    pltpu.sync_copy(src_hbm.at[pl.ds(tbase, CT)], src_t)
    pltpu.sync_copy(dst_hbm.at[pl.ds(tbase, CT)], dst_t)
    pltpu.async_copy(t_hbm.at[src_t], rows0.at[pl.ds(0, CT)], g0).wait()
    pltpu.sync_copy(rows0.at[pl.ds(0, CT)], acc.at[dst_t], add=True)
    plsc.subcore_barrier()

    for j in range(RPT // CE):
        pltpu.sync_copy(acc.at[pl.ds(s * RPT + j * CE, CE)],
                        rows0.at[pl.ds(0, CE)])
        pltpu.sync_copy(rows0.at[pl.ds(0, CE)],
                        sum_out.at[c, pl.ds(s * RPT + j * CE, CE)])


@functools.partial(
    pl.kernel,
    out_type=[jax.ShapeDtypeStruct((E, W), _f32),
              jax.ShapeDtypeStruct((E, W), _f32)],
    mesh=_sc_mesh(),
    scratch_types=[
        pltpu.VMEM((CG,), jnp.int32), pltpu.VMEM((CG,), jnp.int32),
        pltpu.VMEM((CG,), jnp.int32), pltpu.VMEM((CG,), jnp.int32),
        pltpu.VMEM((CG, W), _f32), pltpu.VMEM((CG, W), _f32),
        pltpu.VMEM((CG, W), _f32), pltpu.VMEM((CG, W), _f32),
        pltpu.SemaphoreType.DMA, pltpu.SemaphoreType.DMA,
        pltpu.SemaphoreType.DMA, pltpu.SemaphoreType.DMA,
        pltpu.SemaphoreType.DMA, pltpu.SemaphoreType.DMA,
        pltpu.SemaphoreType.DMA, pltpu.SemaphoreType.DMA,
    ],
)
def _edge_gather(a_hbm, b_hbm, src_hbm, dst_hbm, as_out, bd_out,
                 src0, dst0, src1, dst1, rs0, rd0, rs1, rd1,
                 gs0, gd0, gs1, gd1, ws0, wd0, ws1, wd1):
    c = lax.axis_index("c")
    s = lax.axis_index("s")
    wid = c * NS + s
    ebase = wid * EPW

    def idx_load(i, sv, dv):
        base = ebase + i * CG
        pltpu.sync_copy(src_hbm.at[pl.ds(base, CG)], sv)
        pltpu.sync_copy(dst_hbm.at[pl.ds(base, CG)], dv)

    idx_load(0, src0, dst0)
    pltpu.async_copy(a_hbm.at[src0], rs0, gs0)
    pltpu.async_copy(b_hbm.at[dst0], rd0, gd0)

    def do_chunk(i, sv, dv, rs, rd, gs, gd, ws, wd):
        pltpu.make_async_copy(a_hbm.at[sv], rs, gs).wait()
        pltpu.make_async_copy(b_hbm.at[dv], rd, gd).wait()
        base = ebase + i * CG
        pltpu.async_copy(rs, as_out.at[pl.ds(base, CG)], ws)
        pltpu.async_copy(rd, bd_out.at[pl.ds(base, CG)], wd)

    def wait_writes(i, rs, rd, ws, wd):
        base = ebase + i * CG
        pltpu.make_async_copy(rs, as_out.at[pl.ds(base, CG)], ws).wait()
        pltpu.make_async_copy(rd, bd_out.at[pl.ds(base, CG)], wd).wait()

    def pair(i2, carry):
        a = 2 * i2
        b = a + 1
        idx_load(b, src1, dst1)
        do_chunk(a, src0, dst0, rs0, rd0, gs0, gd0, ws0, wd0)
        pltpu.async_copy(a_hbm.at[src1], rs1, gs1)
        pltpu.async_copy(b_hbm.at[dst1], rd1, gd1)
        wait_writes(a, rs0, rd0, ws0, wd0)

        @pl.when(i2 < NGCHUNK // 2 - 1)
        def _():
            idx_load(a + 2, src0, dst0)
            pltpu.async_copy(a_hbm.at[src0], rs0, gs0)
            pltpu.async_copy(b_hbm.at[dst0], rd0, gd0)

        do_chunk(b, src1, dst1, rs1, rd1, gs1, gd1, ws1, wd1)
        wait_writes(b, rs1, rd1, ws1, wd1)
        return carry

    lax.fori_loop(0, NGCHUNK // 2, pair, 0)


def _proj_body(x_ref, wl_ref, wr_ref, b_ref, t_ref, r_ref):
    xx = x_ref[...]
    p = jnp.dot(xx, wl_ref[...], preferred_element_type=_f32)
    t_ref[...] = jnp.concatenate([p, jnp.ones((N, H), _f32)], axis=-1)
    r_ref[...] = (jnp.dot(xx, wr_ref[...], preferred_element_type=_f32)
                  + b_ref[...])


def _bn_relu(pre, g, be):
    mu = jnp.mean(pre, axis=0, keepdims=True)
    var = jnp.mean((pre - mu) ** 2, axis=0, keepdims=True)
    hh = g * (pre - mu) / jnp.sqrt(var + 1e-5) + be
    return jnp.maximum(hh, 0.0)


def _combine1_body(s_ref, r_ref, g_ref, be_ref, wl_ref, wr_ref,
                   b2_ref, t2_ref, r2_ref, dinv_ref):
    deg = s_ref[0, 0:N, H:H + 1] + s_ref[1, 0:N, H:H + 1]
    dinv = 1.0 / jnp.maximum(deg, 1.0)
    agg = (s_ref[0, 0:N, 0:H] + s_ref[1, 0:N, 0:H]) * dinv
    hh = _bn_relu(agg + r_ref[...], g_ref[...], be_ref[...])
    p2 = jnp.dot(hh, wl_ref[...], preferred_element_type=_f32)
    t2_ref[...] = jnp.concatenate([p2, jnp.ones((N, H), _f32)], axis=-1)
    r2_ref[...] = (jnp.dot(hh, wr_ref[...], preferred_element_type=_f32)
                   + b2_ref[...])
    dinv_ref[...] = dinv


def _combine2_body(s_ref, dinv_ref, r_ref, g_ref, be_ref, w1s_ref,
                   w1d_ref, a_ref, b_ref):
    agg = (s_ref[0, 0:N, 0:H] + s_ref[1, 0:N, 0:H]) * dinv_ref[...]
    hh = _bn_relu(agg + r_ref[...], g_ref[...], be_ref[...])
    a_ref[...] = jnp.dot(hh, w1s_ref[...], preferred_element_type=_f32)
    b_ref[...] = jnp.dot(hh, w1d_ref[...], preferred_element_type=_f32)


def _edge_mlp_body(as_ref, bd_ref, ea_ref, tb_ref, w1e_ref, w1t_ref,
                   b1_ref, w2_ref, b2_ref, w3_ref, b3_ref, out_ref):
    z = as_ref[...] + bd_ref[...]
    z = z + jnp.dot(ea_ref[...], w1e_ref[...], preferred_element_type=_f32)
    z = z + jnp.dot(tb_ref[...], w1t_ref[...], preferred_element_type=_f32)
    z = jnp.maximum(z + b1_ref[...], 0.0)
    z = jnp.maximum(
        jnp.dot(z, w2_ref[...], preferred_element_type=_f32) + b2_ref[...],
        0.0)
    out_ref[...] = (jnp.dot(z, w3_ref[...], preferred_element_type=_f32)
                    + b3_ref[...])


def kernel(x, edge_index, edge_attr, tabular_features, Wl1, Wr1, b1,
           Wl2, Wr2, b2, g1, be1, g2, be2, W1c, b1c, W2c, b2c, W3c, b3c):
    src = edge_index[0]
    dst = edge_index[1]
    zeros_ce = jnp.zeros((CE, W), _f32)

    t1, r1 = pl.pallas_call(
        _proj_body,
        out_shape=[jax.ShapeDtypeStruct((N, W), _f32),
                   jax.ShapeDtypeStruct((N, H), _f32)],
    )(x, Wl1, Wr1, b1.reshape(1, H))
    (s1,) = _segsum(t1, src, dst, zeros_ce)
    t2, r2, dinv = pl.pallas_call(
        _combine1_body,
        out_shape=[jax.ShapeDtypeStruct((N, W), _f32),
                   jax.ShapeDtypeStruct((N, H), _f32),
                   jax.ShapeDtypeStruct((N, 1), _f32)],
    )(s1, r1, g1.reshape(1, H), be1.reshape(1, H), Wl2, Wr2,
      b2.reshape(1, H))

    (s2,) = _segsum(t2, src, dst, zeros_ce)
    a_tab, b_tab = pl.pallas_call(
        _combine2_body,
        out_shape=[jax.ShapeDtypeStruct((N, W), _f32),
                   jax.ShapeDtypeStruct((N, W), _f32)],
    )(s2, dinv, r2, g2.reshape(1, H), be2.reshape(1, H),
      W1c[0:H], W1c[H:2 * H])

    a_s, b_d = _edge_gather(a_tab, b_tab, src, dst)

    w1e = W1c[2 * H:2 * H + 16]
    w1t = W1c[2 * H + 16:]
    BE = 4000
    c_out = W3c.shape[1]
    full = lambda shp: pl.BlockSpec(shp, lambda i: (0, 0))
    out = pl.pallas_call(
        _edge_mlp_body,
        grid=(E // BE,),
        in_specs=[
            pl.BlockSpec((BE, W), lambda i: (i, 0)),
            pl.BlockSpec((BE, W), lambda i: (i, 0)),
            pl.BlockSpec((BE, 16), lambda i: (i, 0)),
            pl.BlockSpec((BE, 8), lambda i: (i, 0)),
            full((16, 128)), full((8, 128)),
            full((1, 128)), full((128, H)), full((1, H)),
            full((H, c_out)), full((1, c_out)),
        ],
        out_specs=pl.BlockSpec((BE, c_out), lambda i: (i, 0)),
        out_shape=jax.ShapeDtypeStruct((E, c_out), _f32),
    )(a_s, b_d, edge_attr, tabular_features, w1e, w1t,
      b1c.reshape(1, 128), W2c, b2c.reshape(1, H), W3c,
      b3c.reshape(1, c_out))
    return out
